# baseline (speedup 1.0000x reference)
import functools

import jax
import jax.numpy as jnp
from jax import lax
from jax.experimental import pallas as pl
from jax.experimental.pallas import tpu as pltpu
from jax.experimental.pallas import tpu_sc as plsc

D = 768
SEQ = 20
NKEY = 4 * 2 * SEQ
NC, NS = 2, 16
NW = NC * NS
CHUNK = 32


def _lut_body(xt_ref, st_ref, tok_ref, pos_ref, seg_t_ref, gamma_ref, beta_ref,
              lut_ref, idx_ref):
    l_iota = lax.broadcasted_iota(jnp.int32, xt_ref.shape, 0)
    idx_ref[...] = xt_ref[...] * (2 * SEQ) + st_ref[...] * SEQ + l_iota

    r = lax.broadcasted_iota(jnp.int32, (NKEY, 1), 0)
    oh_v = (r // (2 * SEQ) == lax.broadcasted_iota(jnp.int32, (NKEY, 4), 1)).astype(jnp.float32)
    oh_s = ((r % (2 * SEQ)) // SEQ == lax.broadcasted_iota(jnp.int32, (NKEY, 2), 1)).astype(jnp.float32)
    oh_l = (r % SEQ == lax.broadcasted_iota(jnp.int32, (NKEY, SEQ), 1)).astype(jnp.float32)
    hi = lax.Precision.HIGHEST
    emb = (jnp.dot(oh_v, tok_ref[...], preferred_element_type=jnp.float32, precision=hi)
           + jnp.dot(oh_s, seg_t_ref[...], preferred_element_type=jnp.float32, precision=hi)
           + jnp.dot(oh_l, pos_ref[0:SEQ, :], preferred_element_type=jnp.float32, precision=hi))
    mean = jnp.mean(emb, axis=-1, keepdims=True)
    c = emb - mean
    var = jnp.mean(c * c, axis=-1, keepdims=True)
    inv = lax.rsqrt(var + 1e-5)
    lut = c * inv * gamma_ref[0, :][None, :] + beta_ref[0, :][None, :]
    lut_ref[...] = lut.astype(jnp.bfloat16)


def _build_lut_and_idx(xt, st, tok_table, pos_table, seg_table, gamma, beta):
    seq_len, b = xt.shape
    d = tok_table.shape[1]
    return pl.pallas_call(
        _lut_body,
        in_specs=[
            pl.BlockSpec(xt.shape, lambda: (0, 0)),
            pl.BlockSpec(st.shape, lambda: (0, 0)),
            pl.BlockSpec(tok_table.shape, lambda: (0, 0)),
            pl.BlockSpec(pos_table.shape, lambda: (0, 0)),
            pl.BlockSpec(seg_table.shape, lambda: (0, 0)),
            pl.BlockSpec((1, d), lambda: (0, 0)),
            pl.BlockSpec((1, d), lambda: (0, 0)),
        ],
        out_specs=[
            pl.BlockSpec((NKEY, d), lambda: (0, 0)),
            pl.BlockSpec((seq_len, b), lambda: (0, 0)),
        ],
        out_shape=[
            jax.ShapeDtypeStruct((NKEY, d), jnp.bfloat16),
            jax.ShapeDtypeStruct((seq_len, b), jnp.int32),
        ],
    )(xt, st, tok_table, pos_table, seg_table,
      gamma.reshape(1, d), beta.reshape(1, d))


def _sc_expand(lut_packed, idx_flat, batch):
    n = idx_flat.shape[0]
    rows_per_w = n // NW
    n_chunks = rows_per_w // CHUNK
    mesh = plsc.VectorSubcoreMesh(core_axis_name="c", subcore_axis_name="s",
                                  num_cores=NC, num_subcores=NS)

    @functools.partial(
        pl.kernel,
        mesh=mesh,
        compiler_params=pltpu.CompilerParams(use_tc_tiling_on_sc=False,
                                             needs_layout_passes=False),
        out_type=jax.ShapeDtypeStruct((SEQ, batch, D), jnp.float32),
        scratch_types=[
            pltpu.VMEM((NKEY, D // 2 + 1), jnp.int32),
            pltpu.VMEM((rows_per_w,), jnp.int32),
            pltpu.VMEM((CHUNK, D + 1), jnp.float32),
            pltpu.VMEM((CHUNK, D + 1), jnp.float32),
            pltpu.SemaphoreType.DMA,
            pltpu.SemaphoreType.DMA,
        ],
    )
    def k(lut_hbm, idx_hbm, out_hbm, lut_v, idx_v, buf0, buf1, ss0, ss1):
        wid = lax.axis_index("s") * NC + lax.axis_index("c")
        base = wid * rows_per_w

        pltpu.sync_copy(lut_hbm, lut_v)
        pltpu.sync_copy(idx_hbm.at[pl.ds(base, rows_per_w)], idx_v)

        bufs = (buf0, buf1)
        ss = (ss0, ss1)
        iota16 = lax.broadcasted_iota(jnp.int32, (16,), 0)

        def assemble(g, p):
            for grp in range(CHUNK // 16):
                keys = idx_v[pl.ds(g * CHUNK + grp * 16, 16)]
                rows16 = iota16 + grp * 16

                @plsc.parallel_loop(0, D // 2, unroll=8)
                def wbody(w):
                    wv = jnp.full((16,), w, jnp.int32)
                    packed = plsc.load_gather(lut_v, [keys, wv])
                    two = plsc.bitcast(packed, jnp.bfloat16)
                    a, b = plsc.unpack(two, format=plsc.PackFormat.INTERLEAVED,
                                       preferred_element_type=jnp.float32)
                    plsc.store_scatter(bufs[p], [rows16, 2 * wv], a)
                    plsc.store_scatter(bufs[p], [rows16, 2 * wv + 1], b)

        def _dst(g):
            r0 = base + g * CHUNK
            return out_hbm.at[r0 // batch, pl.ds(r0 % batch, CHUNK)]

        def store(g, p):
            return pltpu.async_copy(
                bufs[p].at[pl.ds(0, CHUNK), pl.ds(0, D)], _dst(g), ss[p])

        def wait_store(g, p):
            pltpu.make_async_copy(
                bufs[p].at[pl.ds(0, CHUNK), pl.ds(0, D)], _dst(g),
                ss[p]).wait()

        def body(h, carry):
            for p in (0, 1):
                g = 2 * h + p

                @pl.when(h > 0)
                def _():
                    wait_store(g - 2, p)

                assemble(g, p)
                store(g, p)
            return carry

        lax.fori_loop(0, n_chunks // 2, body, 0)
        for p in (0, 1):
            wait_store(n_chunks - 2 + p, p)

    return k(lut_packed, idx_flat)


def kernel(x, seg, tok_table, pos_table, seg_table, gamma, beta):
    b, seq_len = x.shape
    d = tok_table.shape[1]
    lut_bf, idx_t = _build_lut_and_idx(x.T, seg.T, tok_table, pos_table,
                                       seg_table, gamma, beta)
    lut_packed = lax.bitcast_convert_type(
        lut_bf.reshape(NKEY, d // 2, 2), jnp.int32)
    lut_packed = jnp.pad(lut_packed, ((0, 0), (0, 1)))
    out_t = _sc_expand(lut_packed, idx_t.reshape(-1), b)
    return out_t.transpose(1, 0, 2)


# speedup vs baseline: 1.8011x; 1.8011x over previous
import functools

import jax
import jax.numpy as jnp
from jax import lax
from jax.experimental import pallas as pl
from jax.experimental.pallas import tpu as pltpu
from jax.experimental.pallas import tpu_sc as plsc

D = 768
SEQ = 20
NKEY = 4 * 2 * SEQ
NC, NS = 2, 16
NW = NC * NS
CHUNK = 32


def _lut_body(xt_ref, st_ref, tok_ref, pos_ref, seg_t_ref, gamma_ref, beta_ref,
              lut_ref, idx_ref):
    l_iota = lax.broadcasted_iota(jnp.int32, xt_ref.shape, 0)
    idx_ref[...] = xt_ref[...] * (2 * SEQ) + st_ref[...] * SEQ + l_iota

    r = lax.broadcasted_iota(jnp.int32, (NKEY, 1), 0)
    oh_v = (r // (2 * SEQ) == lax.broadcasted_iota(jnp.int32, (NKEY, 4), 1)).astype(jnp.float32)
    oh_s = ((r % (2 * SEQ)) // SEQ == lax.broadcasted_iota(jnp.int32, (NKEY, 2), 1)).astype(jnp.float32)
    oh_l = (r % SEQ == lax.broadcasted_iota(jnp.int32, (NKEY, SEQ), 1)).astype(jnp.float32)
    hi = lax.Precision.HIGHEST
    emb = (jnp.dot(oh_v, tok_ref[...], preferred_element_type=jnp.float32, precision=hi)
           + jnp.dot(oh_s, seg_t_ref[...], preferred_element_type=jnp.float32, precision=hi)
           + jnp.dot(oh_l, pos_ref[0:SEQ, :], preferred_element_type=jnp.float32, precision=hi))
    mean = jnp.mean(emb, axis=-1, keepdims=True)
    c = emb - mean
    var = jnp.mean(c * c, axis=-1, keepdims=True)
    inv = lax.rsqrt(var + 1e-5)
    lut = c * inv * gamma_ref[0, :][None, :] + beta_ref[0, :][None, :]
    lut_ref[...] = lut.astype(jnp.bfloat16)


def _build_lut_and_idx(xt, st, tok_table, pos_table, seg_table, gamma, beta):
    seq_len, b = xt.shape
    d = tok_table.shape[1]
    return pl.pallas_call(
        _lut_body,
        in_specs=[
            pl.BlockSpec(xt.shape, lambda: (0, 0)),
            pl.BlockSpec(st.shape, lambda: (0, 0)),
            pl.BlockSpec(tok_table.shape, lambda: (0, 0)),
            pl.BlockSpec(pos_table.shape, lambda: (0, 0)),
            pl.BlockSpec(seg_table.shape, lambda: (0, 0)),
            pl.BlockSpec((1, d), lambda: (0, 0)),
            pl.BlockSpec((1, d), lambda: (0, 0)),
        ],
        out_specs=[
            pl.BlockSpec((NKEY, d), lambda: (0, 0)),
            pl.BlockSpec((seq_len, b), lambda: (0, 0)),
        ],
        out_shape=[
            jax.ShapeDtypeStruct((NKEY, d), jnp.bfloat16),
            jax.ShapeDtypeStruct((seq_len, b), jnp.int32),
        ],
    )(xt, st, tok_table, pos_table, seg_table,
      gamma.reshape(1, d), beta.reshape(1, d))


def _sc_expand(lut_packed, idx_flat, batch):
    n = idx_flat.shape[0]
    rows_per_w = n // NW
    n_chunks = rows_per_w // CHUNK
    mesh = plsc.VectorSubcoreMesh(core_axis_name="c", subcore_axis_name="s",
                                  num_cores=NC, num_subcores=NS)

    @functools.partial(
        pl.kernel,
        mesh=mesh,
        compiler_params=pltpu.CompilerParams(use_tc_tiling_on_sc=False,
                                             needs_layout_passes=False),
        out_type=jax.ShapeDtypeStruct((SEQ, batch, D), jnp.float32),
        scratch_types=[
            pltpu.VMEM((NKEY, D // 2 + 1), jnp.int32),
            pltpu.VMEM((rows_per_w,), jnp.int32),
            pltpu.VMEM((CHUNK, D + 1), jnp.float32),
            pltpu.VMEM((CHUNK, D + 1), jnp.float32),
            pltpu.SemaphoreType.DMA,
            pltpu.SemaphoreType.DMA,
        ],
    )
    def k(lut_hbm, idx_hbm, out_hbm, lut_v, idx_v, buf0, buf1, ss0, ss1):
        wid = lax.axis_index("s") * NC + lax.axis_index("c")
        base = wid * rows_per_w

        pltpu.sync_copy(lut_hbm, lut_v)
        pltpu.sync_copy(idx_hbm.at[pl.ds(base, rows_per_w)], idx_v)

        bufs = (buf0, buf1)
        ss = (ss0, ss1)
        iota16 = lax.broadcasted_iota(jnp.int32, (16,), 0)

        def assemble(g, p):
            for grp in range(CHUNK // 16):
                keys = idx_v[pl.ds(g * CHUNK + grp * 16, 16)]
                rows16 = iota16 + grp * 16

                @plsc.parallel_loop(0, D // 2, unroll=8)
                def wbody(w):
                    wv = jnp.full((16,), w, jnp.int32) ^ iota16
                    packed = plsc.load_gather(lut_v, [keys, wv])
                    two = plsc.bitcast(packed, jnp.bfloat16)
                    a, b = plsc.unpack(two, format=plsc.PackFormat.INTERLEAVED,
                                       preferred_element_type=jnp.float32)
                    plsc.store_scatter(bufs[p], [rows16, 2 * wv], a)
                    plsc.store_scatter(bufs[p], [rows16, 2 * wv + 1], b)

        def _dst(g):
            r0 = base + g * CHUNK
            return out_hbm.at[r0 // batch, pl.ds(r0 % batch, CHUNK)]

        def store(g, p):
            return pltpu.async_copy(
                bufs[p].at[pl.ds(0, CHUNK), pl.ds(0, D)], _dst(g), ss[p])

        def wait_store(g, p):
            pltpu.make_async_copy(
                bufs[p].at[pl.ds(0, CHUNK), pl.ds(0, D)], _dst(g),
                ss[p]).wait()

        def body(h, carry):
            for p in (0, 1):
                g = 2 * h + p

                @pl.when(h > 0)
                def _():
                    wait_store(g - 2, p)

                assemble(g, p)
                store(g, p)
            return carry

        lax.fori_loop(0, n_chunks // 2, body, 0)
        for p in (0, 1):
            wait_store(n_chunks - 2 + p, p)

    return k(lut_packed, idx_flat)


def kernel(x, seg, tok_table, pos_table, seg_table, gamma, beta):
    b, seq_len = x.shape
    d = tok_table.shape[1]
    lut_bf, idx_t = _build_lut_and_idx(x.T, seg.T, tok_table, pos_table,
                                       seg_table, gamma, beta)
    lut_packed = lax.bitcast_convert_type(
        lut_bf.reshape(NKEY, d // 2, 2), jnp.int32)
    lut_packed = jnp.pad(lut_packed, ((0, 0), (0, 1)))
    out_t = _sc_expand(lut_packed, idx_t.reshape(-1), b)
    return out_t.transpose(1, 0, 2)
